# Initial kernel scaffold; baseline (speedup 1.0000x reference)
import jax
import jax.numpy as jnp
from jax.experimental import pallas as pl


def kernel(x, Wq1, Wk1, Wv1, Wp1, Wq2, Wk2, Wv2, Wp2, Wq3, Wk3, Wv3, Wp3, Wq4, Wk4, Wv4, Wp4, W5, L1, L2, b2, L3, b3):
    raise NotImplementedError("write your pallas kernel here")


# speedup vs baseline: 1.5429x; 1.5429x over previous
import jax
import jax.numpy as jnp
import numpy as np
from jax.experimental import pallas as pl

K = 20
GROUPS = 8
SCALE = 1.0
LEAK = 0.2
EPS = 1e-5
_B = 8
_NU = 256
_R = _B * _NU
_BIG = 1e10


def _dot_nt(a, b):
    return jax.lax.dot_general(a, b, (((1,), (1,)), ((), ())),
                               preferred_element_type=jnp.float32)


def _dot_tn(a, b):
    return jax.lax.dot_general(a, b, (((0,), (0,)), ((), ())),
                               preferred_element_type=jnp.float32,
                               precision=jax.lax.Precision.HIGHEST)


def _bslice(a, b):
    return jax.lax.slice(a, (b * _NU, 0), ((b + 1) * _NU, a.shape[1]))


def _fold_sum(v):
    r, c = v.shape
    if c % 8 != 0:
        v = jnp.concatenate([v, jnp.zeros((r, 8 - c % 8), v.dtype)], axis=1)
        c = v.shape[1]
    t = jax.lax.slice(v, (0, 0), (r, 8))
    for k in range(1, c // 8):
        t = t + jax.lax.slice(v, (0, 8 * k), (r, 8 * k + 8))
    w = 8
    while w > 1:
        w //= 2
        t = jax.lax.slice(t, (0, 0), (r, w)) + \
            jax.lax.slice(t, (0, w), (r, 2 * w))
    return t


def _sel_kernel(h_ref, out_ref):
    h = h_ref[...]
    sq = _fold_sum(h * h)
    ii = jax.lax.broadcasted_iota(jnp.int32, (_NU, _NU), 0)
    jj = jax.lax.broadcasted_iota(jnp.int32, (_NU, _NU), 1)
    eye_nu = (ii == jj).astype(jnp.float32)
    blocks = []
    for b in range(_B):
        hb = _bslice(h, b)
        sqb = _bslice(sq, b)
        inner = _dot_nt(hb, hb)
        sqrow = jax.lax.transpose(sqb, (1, 0))
        blocks.append(sqb - 2.0 * inner + sqrow)
    dist = jnp.concatenate(blocks, axis=0)
    col = jax.lax.broadcasted_iota(jnp.int32, (_R, _NU), 1)
    rowc = jax.lax.rem(jax.lax.broadcasted_iota(jnp.int32, (_R, _NU), 0), _NU)
    d = dist + jnp.where(col == rowc, _BIG, 0.0)
    m1 = jnp.min(d, axis=1, keepdims=True)
    c1 = jnp.min(jnp.where(d == m1, col, _NU), axis=1, keepdims=True)
    d2 = d + jnp.where(col == c1, _BIG, 0.0)
    m2 = jnp.min(d2, axis=1, keepdims=True)
    c2 = jnp.min(jnp.where(d2 == m2, col, _NU), axis=1, keepdims=True)
    d3 = d2 + jnp.where(col == c2, _BIG, 0.0)
    m3 = jnp.min(d3, axis=1, keepdims=True)
    c3 = jnp.min(jnp.where(d3 == m3, col, _NU), axis=1, keepdims=True)
    tie12 = (m1 == m2).astype(jnp.float32)
    tie23 = jnp.logical_and(m3 == m2, m1 != m2).astype(jnp.float32)
    out_ref[...] = jnp.concatenate(
        [c1.astype(jnp.float32), c2.astype(jnp.float32), c3.astype(jnp.float32),
         tie12, tie23, jnp.zeros((_R, 3), jnp.float32)], axis=1)


def _build_idx(sel):
    sel = sel.reshape(_B, _NU, 8)
    c1 = sel[..., 0].astype(jnp.int32)
    c2 = sel[..., 1].astype(jnp.int32)
    c3 = sel[..., 2].astype(jnp.int32)
    t12 = sel[..., 3] > 0.5
    t23 = sel[..., 4] > 0.5
    n_all = jnp.arange(2048, dtype=jnp.int32)
    u = n_all % _NU
    jn = n_all // _NU
    s = jnp.arange(7, dtype=jnp.int32)[None, :]
    selfs = u[:, None] + _NU * (s + (s >= jn[:, None]).astype(jnp.int32))
    selfs = jnp.broadcast_to(selfs[None], (_B, 2048, 7))

    def expand(cb):
        return jnp.tile(cb, (1, 8))

    e1, e2, e3 = expand(c1), expand(c2), expand(c3)
    et12 = jnp.tile(t12, (1, 8))[..., None]
    et23 = jnp.tile(t23, (1, 8))[..., None]
    t = jnp.arange(13, dtype=jnp.int32)[None, None, :]
    normal = jnp.where(t < 8, e1[..., None] + _NU * t,
                       e2[..., None] + _NU * (t - 8))
    inter12 = jnp.where(t % 2 == 0, e1[..., None], e2[..., None]) + _NU * (t // 2)
    r = t - 8
    inter23 = jnp.where(t < 8, e1[..., None] + _NU * t,
                        jnp.where(r % 2 == 0, e2[..., None], e3[..., None])
                        + _NU * (r // 2))
    rest = jnp.where(et12, inter12, jnp.where(et23, inter23, normal))
    return jnp.concatenate([selfs, rest], axis=-1)


def _gather(feats, idx):
    return jax.vmap(lambda f, i: f[i])(feats, idx)


def _get_neighbors(x, idx):
    xt = jnp.transpose(x, (0, 2, 1))
    g = _gather(xt, idx)
    center = xt[:, :, None, :]
    feat = jnp.concatenate([g - center, jnp.broadcast_to(center, g.shape)], axis=-1)
    feat = jnp.transpose(feat, (0, 3, 1, 2))
    return feat, x[:, :, :, None]


def _attn_conv(feat, abs_x, idx, points, wq, wk, wv, wp):
    b, inc, n, k = feat.shape
    o = wq.shape[0]
    q = jnp.einsum('oc,bcnu->bonu', wq, abs_x)
    kf = jnp.einsum('oc,bcnk->bonk', wk, feat)
    v = jnp.einsum('oc,bcnk->bonk', wv, feat)
    pts = jnp.transpose(points, (0, 2, 1))
    pg = _gather(pts, idx)
    rel = jnp.transpose(pg - pts[:, :, None, :], (0, 3, 1, 2))
    kf = kf + jnp.einsum('oc,bcnk->bonk', wp, rel)
    dg = o // GROUPS
    qg = q.reshape(b, GROUPS, dg, n, 1)
    kg = kf.reshape(b, GROUPS, dg, n, k)
    vg = v.reshape(b, GROUPS, dg, n, k)
    logits = jnp.sum(qg * kg, axis=2) * (SCALE / np.sqrt(dg))
    attn = jax.nn.softmax(logits, axis=-1)
    out = jnp.sum(attn[:, :, None, :, :] * vg, axis=-1)
    return out.reshape(b, o, n, 1)


def _bn(x, axes):
    m = jnp.mean(x, axis=axes, keepdims=True)
    v = jnp.var(x, axis=axes, keepdims=True)
    return (x - m) / jnp.sqrt(v + EPS)


def kernel(x, Wq1, Wk1, Wv1, Wp1, Wq2, Wk2, Wv2, Wp2, Wq3, Wk3, Wv3, Wp3,
           Wq4, Wk4, Wv4, Wp4, W5, L1, L2, b2, L3, b3):
    points = jnp.tile(x, (1, 1, 8))
    h = points
    ws = [(Wq1, Wk1, Wv1, Wp1), (Wq2, Wk2, Wv2, Wp2), (Wq3, Wk3, Wv3, Wp3), (Wq4, Wk4, Wv4, Wp4)]
    for wq, wk, wv, wp in ws:
        hu = jnp.transpose(h, (0, 2, 1))[:, :_NU, :].reshape(_R, h.shape[1])
        sel = pl.pallas_call(
            _sel_kernel,
            out_shape=jax.ShapeDtypeStruct((_R, 8), jnp.float32),
        )(hu)
        idx = _build_idx(sel)
        feat, abs_x = _get_neighbors(h, idx)
        h = _attn_conv(feat, abs_x, idx, points, wq, wk, wv, wp)
        h = jax.nn.leaky_relu(_bn(h, (0, 2, 3)), LEAK)[..., 0]
    h = jax.nn.leaky_relu(_bn(jnp.einsum('oc,bcn->bon', W5, h), (0, 2)), LEAK)
    h = jnp.max(h, axis=2)
    h = jax.nn.leaky_relu(_bn(h @ L1.T, (0,)), LEAK)
    h = jax.nn.leaky_relu(_bn(h @ L2.T + b2, (0,)), LEAK)
    return h @ L3.T + b3
